# baseline (speedup 1.0000x reference)
import functools

import jax
import jax.numpy as jnp
from jax import lax
from jax.experimental import pallas as pl
from jax.experimental.pallas import tpu as pltpu
from jax.experimental.pallas import tpu_sc as plsc

NUM_WORKERS = 32
LANES = 16
BT_PER_WORKER = 4


def _make_gather(B0, H, V, D):
    n_bt = B0 // 128
    n_dt = D // 8
    mesh = plsc.VectorSubcoreMesh(core_axis_name="c", subcore_axis_name="s")

    @functools.partial(
        pl.kernel,
        out_type=jax.ShapeDtypeStruct((H, n_dt, n_bt, 8, 128), jnp.float32),
        mesh=mesh,
        scratch_types=[
            pltpu.VMEM((V * D,), jnp.float32),
            pltpu.VMEM((128 * H,), jnp.int32),
            pltpu.VMEM((128 * H,), jnp.int32),
            pltpu.VMEM((n_dt, 8, 128), jnp.float32),
            pltpu.VMEM((n_dt, 8, 128), jnp.float32),
            pltpu.SemaphoreType.DMA,
            pltpu.SemaphoreType.DMA,
            pltpu.SemaphoreType.DMA,
            pltpu.SemaphoreType.DMA,
            pltpu.SemaphoreType.DMA,
        ],
        compiler_params=pltpu.CompilerParams(use_tc_tiling_on_sc=False,
                                             needs_layout_passes=False),
    )
    def gather_kernel(idx_hbm, table_hbm, u_hbm, tab_v, idxb0, idxb1, slab0,
                      slab1, sem_t, sem_i0, sem_i1, sem_s0, sem_s1):
        wid = lax.axis_index("s") * 2 + lax.axis_index("c")
        idxb = (idxb0, idxb1)
        slab = (slab0, slab1)
        sem_i = (sem_i0, sem_i1)
        sem_s = (sem_s0, sem_s1)

        def idx_copy(bl, p):
            bt = wid * BT_PER_WORKER + bl
            return pltpu.make_async_copy(
                idx_hbm.at[pl.ds(bt * 128 * H, 128 * H)], idxb[p], sem_i[p])

        def slab_copy(h, bl, p):
            bt = wid * BT_PER_WORKER + bl
            return pltpu.make_async_copy(slab[p], u_hbm.at[h, :, bt],
                                         sem_s[p])

        pltpu.make_async_copy(table_hbm, tab_v, sem_t).start()
        idx_copy(0, 0).start()
        pltpu.make_async_copy(table_hbm, tab_v, sem_t).wait()

        lane = lax.iota(jnp.int32, LANES)
        row_off = lane * H

        def emit_slab(h, p, q):
            sp = slab[q]
            for g in range(8):
                items16 = plsc.load_gather(
                    idxb[p], [row_off + (g * LANES * H + h)])
                addr = items16 * D
                for dt in range(n_dt):
                    for dr in range(8):
                        vals = plsc.load_gather(tab_v, [addr + (dt * 8 + dr)])
                        sp[dt, dr, pl.ds(g * LANES, LANES)] = vals

        def run_block(bl, p):
            idx_copy(bl, p).wait()

            def h_pair(hp, carry):
                h = hp * 2
                emit_slab(h, p, 0)
                slab_copy(h, bl, 0).start()
                emit_slab(h + 1, p, 1)
                slab_copy(h + 1, bl, 1).start()
                slab_copy(h, bl, 0).wait()
                slab_copy(h + 1, bl, 1).wait()
                return carry

            lax.fori_loop(0, H // 2, h_pair, 0)

        for bl in range(BT_PER_WORKER):
            p = bl % 2
            if bl + 1 < BT_PER_WORKER:
                idx_copy(bl + 1, 1 - p).start()
            run_block(bl, p)

    return gather_kernel


def kernel(items, table):
    B0, H = items.shape
    V, D = table.shape
    idx = items.reshape(-1).astype(jnp.int32)
    u = _make_gather(B0, H, V, D)(idx, table.reshape(-1))
    return u.transpose(2, 4, 0, 1, 3).reshape(B0, H, D)


# speedup vs baseline: 1.6721x; 1.6721x over previous
import functools

import jax
import jax.numpy as jnp
from jax import lax
from jax.experimental import pallas as pl
from jax.experimental.pallas import tpu as pltpu
from jax.experimental.pallas import tpu_sc as plsc

NUM_WORKERS = 32
LANES = 16
BT_PER_WORKER = 4


def _make_gather(B0, H, V, D):
    n_bt = B0 // 128
    n_dt = D // 8
    mesh = plsc.VectorSubcoreMesh(core_axis_name="c", subcore_axis_name="s")

    @functools.partial(
        pl.kernel,
        out_type=jax.ShapeDtypeStruct((H, n_dt, n_bt, 8, 128), jnp.float32),
        mesh=mesh,
        scratch_types=[
            pltpu.VMEM((V * D,), jnp.float32),
            pltpu.VMEM((128 * H,), jnp.int32),
            pltpu.VMEM((128 * H,), jnp.int32),
            pltpu.VMEM((n_dt, 8, 128), jnp.float32),
            pltpu.VMEM((n_dt, 8, 128), jnp.float32),
            pltpu.SemaphoreType.DMA,
            pltpu.SemaphoreType.DMA,
            pltpu.SemaphoreType.DMA,
            pltpu.SemaphoreType.DMA,
            pltpu.SemaphoreType.DMA,
        ],
        compiler_params=pltpu.CompilerParams(use_tc_tiling_on_sc=False,
                                             needs_layout_passes=False),
    )
    def gather_kernel(idx_hbm, table_hbm, u_hbm, tab_v, idxb0, idxb1, slab0,
                      slab1, sem_t, sem_i0, sem_i1, sem_s0, sem_s1):
        wid = lax.axis_index("s") * 2 + lax.axis_index("c")
        idxb = (idxb0, idxb1)
        slab = (slab0, slab1)
        sem_i = (sem_i0, sem_i1)
        sem_s = (sem_s0, sem_s1)

        def idx_copy(bl, p):
            bt = wid * BT_PER_WORKER + bl
            return pltpu.make_async_copy(
                idx_hbm.at[pl.ds(bt * 128 * H, 128 * H)], idxb[p], sem_i[p])

        def slab_copy(h, bl, p):
            bt = wid * BT_PER_WORKER + bl
            return pltpu.make_async_copy(slab[p], u_hbm.at[h, :, bt],
                                         sem_s[p])

        pltpu.make_async_copy(table_hbm, tab_v, sem_t).start()
        idx_copy(0, 0).start()
        pltpu.make_async_copy(table_hbm, tab_v, sem_t).wait()

        lane = lax.iota(jnp.int32, LANES)
        row_off = lane * H

        def emit_slab(h, p, q):
            sp = slab[q]
            for g in range(8):
                items16 = plsc.load_gather(
                    idxb[p], [row_off + (g * LANES * H + h)])
                addr = items16 * D
                vals = [
                    plsc.load_gather(tab_v, [addr + d]) for d in range(D)
                ]
                for dt in range(n_dt):
                    for dr in range(8):
                        sp[dt, dr, pl.ds(g * LANES, LANES)] = vals[dt * 8 + dr]

        def run_block(bl, p):
            idx_copy(bl, p).wait()

            def h_pair(hp, carry):
                h = hp * 2
                emit_slab(h, p, 0)
                slab_copy(h, bl, 0).start()
                emit_slab(h + 1, p, 1)
                slab_copy(h + 1, bl, 1).start()
                slab_copy(h, bl, 0).wait()
                slab_copy(h + 1, bl, 1).wait()
                return carry

            lax.fori_loop(0, H // 2, h_pair, 0)

        for bl in range(BT_PER_WORKER):
            p = bl % 2
            if bl + 1 < BT_PER_WORKER:
                idx_copy(bl + 1, 1 - p).start()
            run_block(bl, p)

    return gather_kernel


def kernel(items, table):
    B0, H = items.shape
    V, D = table.shape
    idx = items.reshape(-1).astype(jnp.int32)
    u = _make_gather(B0, H, V, D)(idx, table.reshape(-1))
    return u.transpose(2, 4, 0, 1, 3).reshape(B0, H, D)


# speedup vs baseline: 1.7615x; 1.0535x over previous
import functools

import jax
import jax.numpy as jnp
from jax import lax
from jax.experimental import pallas as pl
from jax.experimental.pallas import tpu as pltpu
from jax.experimental.pallas import tpu_sc as plsc

NUM_WORKERS = 32
LANES = 16
BT_PER_WORKER = 4


def _make_gather(B0, H, V, D):
    n_bt = B0 // 128
    n_dt = D // 8
    mesh = plsc.VectorSubcoreMesh(core_axis_name="c", subcore_axis_name="s")

    @functools.partial(
        pl.kernel,
        out_type=jax.ShapeDtypeStruct((H, n_dt, n_bt, 8, 128), jnp.float32),
        mesh=mesh,
        scratch_types=[
            pltpu.VMEM((V * D,), jnp.float32),
            pltpu.VMEM((128 * H,), jnp.int32),
            pltpu.VMEM((128 * H,), jnp.int32),
            pltpu.VMEM((n_dt, 8, 128), jnp.float32),
            pltpu.VMEM((n_dt, 8, 128), jnp.float32),
            pltpu.SemaphoreType.DMA,
            pltpu.SemaphoreType.DMA,
            pltpu.SemaphoreType.DMA,
            pltpu.SemaphoreType.DMA,
            pltpu.SemaphoreType.DMA,
        ],
        compiler_params=pltpu.CompilerParams(use_tc_tiling_on_sc=False,
                                             needs_layout_passes=False),
    )
    def gather_kernel(idx_hbm, table_hbm, u_hbm, tab_v, idxb0, idxb1, slab0,
                      slab1, sem_t, sem_i0, sem_i1, sem_s0, sem_s1):
        wid = lax.axis_index("s") * 2 + lax.axis_index("c")
        idxb = (idxb0, idxb1)
        slab = (slab0, slab1)
        sem_i = (sem_i0, sem_i1)
        sem_s = (sem_s0, sem_s1)

        def idx_copy(bl, p):
            bt = wid * BT_PER_WORKER + bl
            return pltpu.make_async_copy(
                idx_hbm.at[pl.ds(bt * 128 * H, 128 * H)], idxb[p], sem_i[p])

        def slab_copy(h, bl, p):
            bt = wid * BT_PER_WORKER + bl
            return pltpu.make_async_copy(slab[p], u_hbm.at[h, :, bt],
                                         sem_s[p])

        pltpu.make_async_copy(table_hbm, tab_v, sem_t).start()
        idx_copy(0, 0).start()
        pltpu.make_async_copy(table_hbm, tab_v, sem_t).wait()

        lane = lax.iota(jnp.int32, LANES)
        row_off = lane * H

        def emit_slab(h, p, q):
            sp = slab[q]
            for g in range(8):
                items16 = plsc.load_gather(
                    idxb[p], [row_off + (g * LANES * H + h)])
                addr = items16 * D
                vals = [
                    plsc.load_gather(tab_v, [addr + d]) for d in range(D)
                ]
                for dt in range(n_dt):
                    for dr in range(8):
                        sp[dt, dr, pl.ds(g * LANES, LANES)] = vals[dt * 8 + dr]

        def run_block(bl, p):
            idx_copy(bl, p).wait()
            emit_slab(0, p, 0)
            slab_copy(0, bl, 0).start()
            emit_slab(1, p, 1)
            slab_copy(1, bl, 1).start()

            def h_pair(hp, carry):
                h = hp * 2
                slab_copy(h - 2, bl, 0).wait()
                emit_slab(h, p, 0)
                slab_copy(h, bl, 0).start()
                slab_copy(h - 1, bl, 1).wait()
                emit_slab(h + 1, p, 1)
                slab_copy(h + 1, bl, 1).start()
                return carry

            lax.fori_loop(1, H // 2, h_pair, 0)
            slab_copy(H - 2, bl, 0).wait()
            slab_copy(H - 1, bl, 1).wait()

        def bl_pair(j, carry):
            bl0 = 2 * j
            idx_copy(bl0 + 1, 1).start()
            run_block(bl0, 0)

            @pl.when(j + 1 < BT_PER_WORKER // 2)
            def _():
                idx_copy(bl0 + 2, 0).start()

            run_block(bl0 + 1, 1)
            return carry

        lax.fori_loop(0, BT_PER_WORKER // 2, bl_pair, 0)

    return gather_kernel


def kernel(items, table):
    B0, H = items.shape
    V, D = table.shape
    idx = items.reshape(-1).astype(jnp.int32)
    u = _make_gather(B0, H, V, D)(idx, table.reshape(-1))
    return u.transpose(2, 4, 0, 1, 3).reshape(B0, H, D)


# speedup vs baseline: 8.3256x; 4.7265x over previous
import functools

import jax
import jax.numpy as jnp
from jax import lax
from jax.experimental import pallas as pl
from jax.experimental.pallas import tpu as pltpu
from jax.experimental.pallas import tpu_sc as plsc

NUM_WORKERS = 32
LANES = 16
BT_PER_WORKER = 4


def _make_gather(B0, H, V, D):
    n_bt = B0 // 128
    n_dt = D // 8
    mesh = plsc.VectorSubcoreMesh(core_axis_name="c", subcore_axis_name="s")

    @functools.partial(
        pl.kernel,
        out_type=jax.ShapeDtypeStruct((H, n_dt, n_bt, 8, 128), jnp.float32),
        mesh=mesh,
        scratch_types=[
            pltpu.VMEM((V * D,), jnp.float32),
            pltpu.VMEM((H, 128), jnp.int32),
            pltpu.VMEM((H, 128), jnp.int32),
            pltpu.VMEM((n_dt, 8, 128), jnp.float32),
            pltpu.VMEM((n_dt, 8, 128), jnp.float32),
            pltpu.SemaphoreType.DMA,
            pltpu.SemaphoreType.DMA,
            pltpu.SemaphoreType.DMA,
            pltpu.SemaphoreType.DMA,
            pltpu.SemaphoreType.DMA,
        ],
        compiler_params=pltpu.CompilerParams(use_tc_tiling_on_sc=False,
                                             needs_layout_passes=False),
    )
    def gather_kernel(idx_hbm, table_hbm, u_hbm, tab_v, idxb0, idxb1, slab0,
                      slab1, sem_t, sem_i0, sem_i1, sem_s0, sem_s1):
        wid = lax.axis_index("s") * 2 + lax.axis_index("c")
        idxb = (idxb0, idxb1)
        slab = (slab0, slab1)
        sem_i = (sem_i0, sem_i1)
        sem_s = (sem_s0, sem_s1)

        def idx_copy(bl, p):
            bt = wid * BT_PER_WORKER + bl
            return pltpu.make_async_copy(
                idx_hbm.at[:, pl.ds(bt * 128, 128)], idxb[p], sem_i[p])

        def slab_copy(h, bl, p):
            bt = wid * BT_PER_WORKER + bl
            return pltpu.make_async_copy(slab[p], u_hbm.at[h, :, bt],
                                         sem_s[p])

        pltpu.make_async_copy(table_hbm, tab_v, sem_t).start()
        idx_copy(0, 0).start()
        pltpu.make_async_copy(table_hbm, tab_v, sem_t).wait()

        def emit_slab(h, p, q):
            sp = slab[q]
            for g in range(8):
                items16 = idxb[p][h, pl.ds(g * LANES, LANES)]
                vals = [
                    plsc.load_gather(tab_v, [items16 + d * V])
                    for d in range(D)
                ]
                for dt in range(n_dt):
                    for dr in range(8):
                        sp[dt, dr, pl.ds(g * LANES, LANES)] = vals[dt * 8 + dr]

        def run_block(bl, p):
            idx_copy(bl, p).wait()
            emit_slab(0, p, 0)
            slab_copy(0, bl, 0).start()
            emit_slab(1, p, 1)
            slab_copy(1, bl, 1).start()

            def h_pair(hp, carry):
                h = hp * 2
                slab_copy(h - 2, bl, 0).wait()
                emit_slab(h, p, 0)
                slab_copy(h, bl, 0).start()
                slab_copy(h - 1, bl, 1).wait()
                emit_slab(h + 1, p, 1)
                slab_copy(h + 1, bl, 1).start()
                return carry

            lax.fori_loop(1, H // 2, h_pair, 0)
            slab_copy(H - 2, bl, 0).wait()
            slab_copy(H - 1, bl, 1).wait()

        def bl_pair(j, carry):
            bl0 = 2 * j
            idx_copy(bl0 + 1, 1).start()
            run_block(bl0, 0)

            @pl.when(j + 1 < BT_PER_WORKER // 2)
            def _():
                idx_copy(bl0 + 2, 0).start()

            run_block(bl0 + 1, 1)
            return carry

        lax.fori_loop(0, BT_PER_WORKER // 2, bl_pair, 0)

    return gather_kernel


def kernel(items, table):
    B0, H = items.shape
    V, D = table.shape
    idx = items.T.astype(jnp.int32)
    u = _make_gather(B0, H, V, D)(idx, table.T.reshape(-1))
    return u.transpose(2, 4, 0, 1, 3).reshape(B0, H, D)


# speedup vs baseline: 8.7285x; 1.0484x over previous
import functools

import jax
import jax.numpy as jnp
from jax import lax
from jax.experimental import pallas as pl
from jax.experimental.pallas import tpu as pltpu
from jax.experimental.pallas import tpu_sc as plsc

NUM_WORKERS = 32
LANES = 16
BT_PER_WORKER = 4


def _make_gather(B0, H, V, D):
    n_bt = B0 // 128
    n_dt = D // 8
    mesh = plsc.VectorSubcoreMesh(core_axis_name="c", subcore_axis_name="s")

    @functools.partial(
        pl.kernel,
        out_type=jax.ShapeDtypeStruct((H, n_dt, n_bt, 8, 128), jnp.float32),
        mesh=mesh,
        scratch_types=[
            pltpu.VMEM((V * D,), jnp.float32),
            pltpu.VMEM((H // 8, 8, 128), jnp.int32),
            pltpu.VMEM((H // 8, 8, 128), jnp.int32),
            pltpu.VMEM((n_dt, 8, 128), jnp.float32),
            pltpu.VMEM((n_dt, 8, 128), jnp.float32),
            pltpu.SemaphoreType.DMA,
            pltpu.SemaphoreType.DMA,
            pltpu.SemaphoreType.DMA,
            pltpu.SemaphoreType.DMA,
            pltpu.SemaphoreType.DMA,
        ],
        compiler_params=pltpu.CompilerParams(use_tc_tiling_on_sc=False,
                                             needs_layout_passes=False),
    )
    def gather_kernel(idx_hbm, table_hbm, u_hbm, tab_v, idxb0, idxb1, slab0,
                      slab1, sem_t, sem_i0, sem_i1, sem_s0, sem_s1):
        wid = lax.axis_index("s") * 2 + lax.axis_index("c")
        idxb = (idxb0, idxb1)
        slab = (slab0, slab1)
        sem_i = (sem_i0, sem_i1)
        sem_s = (sem_s0, sem_s1)

        def idx_copy(bl, p):
            bt = wid * BT_PER_WORKER + bl
            return pltpu.make_async_copy(idx_hbm.at[:, bt], idxb[p],
                                         sem_i[p])

        def slab_copy(h, bl, p):
            bt = wid * BT_PER_WORKER + bl
            return pltpu.make_async_copy(slab[p], u_hbm.at[h, :, bt],
                                         sem_s[p])

        pltpu.make_async_copy(table_hbm, tab_v, sem_t).start()
        idx_copy(0, 0).start()
        pltpu.make_async_copy(table_hbm, tab_v, sem_t).wait()

        def emit_slab(h, p, q):
            sp = slab[q]
            for g in range(8):
                items16 = idxb[p][h // 8, h % 8, pl.ds(g * LANES, LANES)]
                vals = [
                    plsc.load_gather(tab_v, [items16 + d * V])
                    for d in range(D)
                ]
                for dt in range(n_dt):
                    for dr in range(8):
                        sp[dt, dr, pl.ds(g * LANES, LANES)] = vals[dt * 8 + dr]

        def run_block(bl, p):
            idx_copy(bl, p).wait()
            emit_slab(0, p, 0)
            slab_copy(0, bl, 0).start()
            emit_slab(1, p, 1)
            slab_copy(1, bl, 1).start()

            def h_pair(hp, carry):
                h = hp * 2
                slab_copy(h - 2, bl, 0).wait()
                emit_slab(h, p, 0)
                slab_copy(h, bl, 0).start()
                slab_copy(h - 1, bl, 1).wait()
                emit_slab(h + 1, p, 1)
                slab_copy(h + 1, bl, 1).start()
                return carry

            lax.fori_loop(1, H // 2, h_pair, 0)
            slab_copy(H - 2, bl, 0).wait()
            slab_copy(H - 1, bl, 1).wait()

        def bl_pair(j, carry):
            bl0 = 2 * j
            idx_copy(bl0 + 1, 1).start()
            run_block(bl0, 0)

            @pl.when(j + 1 < BT_PER_WORKER // 2)
            def _():
                idx_copy(bl0 + 2, 0).start()

            run_block(bl0 + 1, 1)
            return carry

        lax.fori_loop(0, BT_PER_WORKER // 2, bl_pair, 0)

    return gather_kernel


def kernel(items, table):
    B0, H = items.shape
    V, D = table.shape
    idx = items.astype(jnp.int32).reshape(B0 // 128, 128, H // 8, 8)
    idx = idx.transpose(2, 0, 3, 1)
    u = _make_gather(B0, H, V, D)(idx, table.T.reshape(-1))
    return u.transpose(2, 4, 0, 1, 3).reshape(B0, H, D)
